# Initial kernel scaffold; baseline (speedup 1.0000x reference)
import jax
import jax.numpy as jnp
from jax.experimental import pallas as pl


def kernel(mentions, W, b, first):
    raise NotImplementedError("write your pallas kernel here")


# speedup vs baseline: 4.8968x; 4.8968x over previous
import functools

import jax
import jax.numpy as jnp
from jax.experimental import pallas as pl
from jax.experimental.pallas import tpu as pltpu

N = 8192
D = 128
K = 50
BR = 256
BC = 256
MASK_BASE = -1.0e30
MASK_STEP = -1.0e26
EXTRACTED = -3.0e30
NEG_INF = float("-inf")


def _topk_kernel(m_blk_ref, m_full_ref, w_ref, b_ref,
                 scores_out_ref, idx_out_ref, buf_ref):
    i = pl.program_id(0)
    n_col_blocks = m_full_ref.shape[0] // BC

    w = jax.lax.dot_general(
        m_blk_ref[...], w_ref[...],
        dimension_numbers=(((1,), (1,)), ((), ())),
        preferred_element_type=jnp.float32) + b_ref[...]

    row_ids = jax.lax.broadcasted_iota(jnp.int32, (BR, BC), 0) + i * BR

    def col_block(c, _):
        m_c = m_full_ref[pl.ds(c * BC, BC), :]
        s = jax.lax.dot_general(
            w, m_c,
            dimension_numbers=(((1,), (1,)), ((), ())),
            preferred_element_type=jnp.float32)
        col_ids = jax.lax.broadcasted_iota(jnp.int32, (BR, BC), 1) + c * BC
        maskval = MASK_BASE + col_ids.astype(jnp.float32) * MASK_STEP
        s = jnp.where(col_ids < row_ids, s, maskval)
        buf_ref[:, pl.ds(c * BC, BC)] = s
        return 0

    jax.lax.fori_loop(0, i + 1, col_block, 0)

    def fill_block(c, _):
        col_ids = jax.lax.broadcasted_iota(jnp.int32, (BR, BC), 1) + c * BC
        buf_ref[:, pl.ds(c * BC, BC)] = (
            MASK_BASE + col_ids.astype(jnp.float32) * MASK_STEP)
        return 0

    jax.lax.fori_loop(i + 1, n_col_blocks, fill_block, 0)

    n_cols = buf_ref.shape[1]
    col_iota = jax.lax.broadcasted_iota(jnp.int32, (BR, n_cols), 1)
    out_iota = jax.lax.broadcasted_iota(jnp.int32, (BR, K), 1)

    def extract(t, carry):
        s_acc, i_acc = carry
        buf = buf_ref[...]
        m = jnp.max(buf, axis=1, keepdims=True)
        eq = buf == m
        idx = jnp.min(jnp.where(eq, col_iota, n_cols), axis=1,
                      keepdims=True)
        buf_ref[...] = jnp.where(col_iota == idx, EXTRACTED, buf)
        mval = jnp.where(m > MASK_BASE * 0.5, m, NEG_INF)
        s_acc = jnp.where(out_iota == t, mval, s_acc)
        i_acc = jnp.where(out_iota == t, idx, i_acc)
        return (s_acc, i_acc)

    s_acc, i_acc = jax.lax.fori_loop(
        0, K, extract,
        (jnp.full((BR, K), NEG_INF, jnp.float32),
         jnp.zeros((BR, K), jnp.int32)))
    scores_out_ref[...] = s_acc
    idx_out_ref[...] = i_acc


@jax.jit
def kernel(mentions, W, b, first):
    n = mentions.shape[0]
    grid = (n // BR,)
    scores, idx = pl.pallas_call(
        _topk_kernel,
        grid=grid,
        in_specs=[
            pl.BlockSpec((BR, D), lambda i: (i, 0)),
            pl.BlockSpec((n, D), lambda i: (0, 0)),
            pl.BlockSpec((D, D), lambda i: (0, 0)),
            pl.BlockSpec((1, D), lambda i: (0, 0)),
        ],
        out_specs=[
            pl.BlockSpec((BR, K), lambda i: (i, 0)),
            pl.BlockSpec((BR, K), lambda i: (i, 0)),
        ],
        out_shape=[
            jax.ShapeDtypeStruct((n, K), jnp.float32),
            jax.ShapeDtypeStruct((n, K), jnp.int32),
        ],
        scratch_shapes=[pltpu.VMEM((BR, n), jnp.float32)],
        compiler_params=pltpu.CompilerParams(
            dimension_semantics=("arbitrary",)),
    )(mentions, mentions, W, b.reshape(1, D))
    return scores, idx


# speedup vs baseline: 5.4730x; 1.1177x over previous
import functools

import jax
import jax.numpy as jnp
from jax import lax
from jax.experimental import pallas as pl
from jax.experimental.pallas import tpu as pltpu
from jax.experimental.pallas import tpu_sc as plsc

N = 8192
D = 128
K = 50
KPAD = 64
BR = 256
BC = 256
MASK_BASE = -1.0e30
MASK_STEP = -1.0e26
NEG_SENT = -3.0e38
NEG_INF = float("-inf")

NC = 2
NS = 16
NW = NC * NS
LANES = 16
POOL = 80
REBUILD_AT = 48


def _scores_kernel(m_blk_ref, m_full_ref, w_ref, b_ref, out_ref):
    i = pl.program_id(0)

    w = lax.dot_general(
        m_blk_ref[...], w_ref[...],
        dimension_numbers=(((1,), (1,)), ((), ())),
        preferred_element_type=jnp.float32) + b_ref[...]

    row_ids = lax.broadcasted_iota(jnp.int32, (BR, BC), 0) + i * BR

    def col_block(c, _):
        m_c = m_full_ref[pl.ds(c * BC, BC), :]
        s = lax.dot_general(
            w, m_c,
            dimension_numbers=(((1,), (1,)), ((), ())),
            preferred_element_type=jnp.float32)
        col_ids = lax.broadcasted_iota(jnp.int32, (BR, BC), 1) + c * BC
        maskval = MASK_BASE + col_ids.astype(jnp.float32) * MASK_STEP
        out_ref[:, pl.ds(c * BC, BC)] = jnp.where(col_ids < row_ids, s,
                                                  maskval)
        return 0

    lax.fori_loop(0, i + 1, col_block, 0)


def _compute_scores(mentions, W, b):
    n = mentions.shape[0]
    return pl.pallas_call(
        _scores_kernel,
        grid=(n // BR,),
        in_specs=[
            pl.BlockSpec((BR, D), lambda i: (i, 0)),
            pl.BlockSpec((n, D), lambda i: (0, 0)),
            pl.BlockSpec((D, D), lambda i: (0, 0)),
            pl.BlockSpec((1, D), lambda i: (0, 0)),
        ],
        out_specs=pl.BlockSpec((BR, n), lambda i: (i, 0)),
        out_shape=jax.ShapeDtypeStruct((n, n), jnp.float32),
        compiler_params=pltpu.CompilerParams(
            dimension_semantics=("arbitrary",)),
    )(mentions, mentions, W, b.reshape(1, D))


def _merge_into_list(lk_ref, lv_ref, a_k, a_v):
    for t in range(4):
        b_k = lk_ref[t]
        b_v = lv_ref[t]
        rb_k = lax.rev(b_k, (0,))
        rb_v = lax.rev(b_v, (0,))
        take_a = a_k >= rb_k
        hi_k = jnp.where(take_a, a_k, rb_k)
        hi_v = jnp.where(take_a, a_v, rb_v)
        lo_k = jnp.where(take_a, rb_k, a_k)
        lo_v = jnp.where(take_a, rb_v, a_v)
        nk, nv = plsc.sort_key_val(hi_k, hi_v, descending=True)
        lk_ref[t] = nk
        lv_ref[t] = nv
        a_k, a_v = plsc.sort_key_val(lo_k, lo_v, descending=True)


def _gather16(x, idx):
    dnums = lax.GatherDimensionNumbers(
        offset_dims=(), collapsed_slice_dims=(0,), start_index_map=(0,))
    return lax.gather(x, idx[:, None], dnums, slice_sizes=(1,),
                      mode=lax.GatherScatterMode.PROMISE_IN_BOUNDS)


def _splat_last(x):
    return _gather16(x, jnp.full((LANES,), LANES - 1, jnp.int32))


def _lane_sum(m):
    lane_iota = lax.iota(jnp.int32, LANES)
    s = m
    for k in (8, 4, 2, 1):
        s = s + _gather16(s, lane_iota ^ k)
    return s


def _lane_any(m):
    lane_iota = lax.iota(jnp.int32, LANES)
    s = m.astype(jnp.int32)
    for k in (8, 4, 2, 1):
        s = s | _gather16(s, lane_iota ^ k)
    return s[0] > 0


def _rebuild(lk_ref, lv_ref, pool_v, pool_i, cnt):
    lane_iota = lax.iota(jnp.int32, LANES)
    for j in range(4):
        off = j * LANES
        pv = pool_v[pl.ds(off, LANES)]
        pi = pool_i[pl.ds(off, LANES)]
        valid = (lane_iota + off) < cnt
        pvm = jnp.where(valid, pv, NEG_SENT)
        sk, sv = plsc.sort_key_val(pvm, pi, descending=True)
        _merge_into_list(lk_ref, lv_ref, sk, sv)
    return _splat_last(lk_ref[3])


def _march_row(row, buf_ref, pool_v, pool_i, lk_ref, lv_ref,
               stag_v, stag_i, out_s_hbm, out_i_hbm):
    nvec = (jnp.maximum(row, 56) + 15) // 16
    for j in range(4):
        lk_ref[j] = jnp.full((LANES,), NEG_SENT, jnp.float32)
        lv_ref[j] = jnp.zeros((LANES,), jnp.int32)

    lane_iota = lax.iota(jnp.int32, LANES)

    def scan_body(i, carry):
        tau, cnt = carry
        v = buf_ref[pl.ds(i * LANES, LANES)]
        m = v > tau

        def append_branch():
            nc = _lane_sum(m.astype(jnp.int32))
            idxv = lane_iota + i * LANES
            vm = jnp.where(m, v, NEG_SENT)
            sk, sv = plsc.sort_key_val(vm, idxv, descending=True)
            slots = cnt + lane_iota
            plsc.store_scatter(pool_v, [slots], sk)
            plsc.store_scatter(pool_i, [slots], sv)
            cnt2 = cnt + nc

            def rebuild_branch():
                tau2 = _rebuild(lk_ref, lv_ref, pool_v, pool_i, cnt2)
                return tau2, jnp.zeros((LANES,), jnp.int32)

            return lax.cond(cnt2[0] >= REBUILD_AT, rebuild_branch,
                            lambda: (tau, cnt2))

        return lax.cond(_lane_any(m), append_branch, lambda: (tau, cnt))

    tau0 = jnp.full((LANES,), NEG_SENT, jnp.float32)
    cnt0 = jnp.zeros((LANES,), jnp.int32)
    _, cnt = lax.fori_loop(0, nvec, scan_body, (tau0, cnt0))
    _rebuild(lk_ref, lv_ref, pool_v, pool_i, cnt)

    for j in range(4):
        k = lk_ref[j]
        outv = jnp.where(k < -1.0e29, NEG_INF, k)
        stag_v[pl.ds(j * LANES, LANES)] = outv
        stag_i[pl.ds(j * LANES, LANES)] = lv_ref[j]
    pltpu.sync_copy(stag_v, out_s_hbm.at[row])
    pltpu.sync_copy(stag_i, out_i_hbm.at[row])


def _sc_topk_body(scores_hbm, out_s_hbm, out_i_hbm,
                  rowbuf_a, rowbuf_b, pool_v, pool_i, lk_ref, lv_ref,
                  stag_v, stag_i, sem):
    wid = lax.axis_index("s") * NC + lax.axis_index("c")
    rows_per_worker = N // NW

    pltpu.async_copy(scores_hbm.at[wid], rowbuf_a, sem)

    def pair_body(q, _):
        row_a = q * 2 * NW + wid
        row_b = row_a + NW
        pltpu.make_async_copy(scores_hbm.at[row_a], rowbuf_a, sem).wait()
        pltpu.async_copy(scores_hbm.at[row_b], rowbuf_b, sem)
        _march_row(row_a, rowbuf_a, pool_v, pool_i, lk_ref, lv_ref,
                   stag_v, stag_i, out_s_hbm, out_i_hbm)
        pltpu.make_async_copy(scores_hbm.at[row_b], rowbuf_b, sem).wait()

        @pl.when(q + 1 < rows_per_worker // 2)
        def _():
            pltpu.async_copy(scores_hbm.at[row_a + 2 * NW], rowbuf_a, sem)

        _march_row(row_b, rowbuf_b, pool_v, pool_i, lk_ref, lv_ref,
                   stag_v, stag_i, out_s_hbm, out_i_hbm)
        return 0

    lax.fori_loop(0, rows_per_worker // 2, pair_body, 0)


_sc_topk = functools.partial(
    pl.kernel,
    out_type=[
        jax.ShapeDtypeStruct((N, KPAD), jnp.float32),
        jax.ShapeDtypeStruct((N, KPAD), jnp.int32),
    ],
    mesh=plsc.VectorSubcoreMesh(core_axis_name="c", subcore_axis_name="s"),
    compiler_params=pltpu.CompilerParams(needs_layout_passes=False),
    scratch_types=[
        pltpu.VMEM((N,), jnp.float32),
        pltpu.VMEM((N,), jnp.float32),
        pltpu.VMEM((POOL,), jnp.float32),
        pltpu.VMEM((POOL,), jnp.int32),
        pltpu.VMEM((4, LANES), jnp.float32),
        pltpu.VMEM((4, LANES), jnp.int32),
        pltpu.VMEM((KPAD,), jnp.float32),
        pltpu.VMEM((KPAD,), jnp.int32),
        pltpu.SemaphoreType.DMA,
    ],
)(_sc_topk_body)


@jax.jit
def kernel(mentions, W, b, first):
    scores = _compute_scores(mentions, W, b)
    top_s, top_i = _sc_topk(scores)
    return top_s[:, :K], top_i[:, :K]


# speedup vs baseline: 7.1559x; 1.3075x over previous
import functools

import jax
import jax.numpy as jnp
from jax import lax
from jax.experimental import pallas as pl
from jax.experimental.pallas import tpu as pltpu
from jax.experimental.pallas import tpu_sc as plsc

N = 8192
D = 128
K = 50
KPAD = 64
BR = 256
BC = 256
MASK_BASE = -1.0e30
MASK_STEP = -1.0e26
NEG_SENT = -3.0e38
NEG_INF = float("-inf")

NC = 2
NS = 16
NW = NC * NS
LANES = 16
POOL = 128
REBUILD_AT = 48


def _scores_kernel(m_blk_ref, m_full_ref, w_ref, b_ref, out_ref):
    i = pl.program_id(0)

    w = lax.dot_general(
        m_blk_ref[...], w_ref[...],
        dimension_numbers=(((1,), (1,)), ((), ())),
        preferred_element_type=jnp.float32) + b_ref[...]

    row_ids = lax.broadcasted_iota(jnp.int32, (BR, BC), 0) + i * BR

    def col_block(c, _):
        m_c = m_full_ref[pl.ds(c * BC, BC), :]
        s = lax.dot_general(
            w, m_c,
            dimension_numbers=(((1,), (1,)), ((), ())),
            preferred_element_type=jnp.float32)
        col_ids = lax.broadcasted_iota(jnp.int32, (BR, BC), 1) + c * BC
        maskval = MASK_BASE + col_ids.astype(jnp.float32) * MASK_STEP
        out_ref[:, pl.ds(c * BC, BC)] = jnp.where(col_ids < row_ids, s,
                                                  maskval)
        return 0

    n_col_blocks = m_full_ref.shape[0] // BC
    lax.fori_loop(0, jnp.minimum(i + 2, n_col_blocks), col_block, 0)


def _compute_scores(mentions, W, b):
    n = mentions.shape[0]
    return pl.pallas_call(
        _scores_kernel,
        grid=(n // BR,),
        in_specs=[
            pl.BlockSpec((BR, D), lambda i: (i, 0)),
            pl.BlockSpec((n, D), lambda i: (0, 0)),
            pl.BlockSpec((D, D), lambda i: (0, 0)),
            pl.BlockSpec((1, D), lambda i: (0, 0)),
        ],
        out_specs=pl.BlockSpec((BR, n), lambda i: (i, 0)),
        out_shape=jax.ShapeDtypeStruct((n, n), jnp.float32),
        compiler_params=pltpu.CompilerParams(
            dimension_semantics=("arbitrary",)),
    )(mentions, mentions, W, b.reshape(1, D))


def _merge_into_list(lk_ref, lv_ref, a_k, a_v):
    for t in range(4):
        b_k = lk_ref[t]
        b_v = lv_ref[t]
        rb_k = lax.rev(b_k, (0,))
        rb_v = lax.rev(b_v, (0,))
        take_a = a_k >= rb_k
        hi_k = jnp.where(take_a, a_k, rb_k)
        hi_v = jnp.where(take_a, a_v, rb_v)
        lo_k = jnp.where(take_a, rb_k, a_k)
        lo_v = jnp.where(take_a, rb_v, a_v)
        nk, nv = plsc.sort_key_val(hi_k, hi_v, descending=True)
        lk_ref[t] = nk
        lv_ref[t] = nv
        a_k, a_v = plsc.sort_key_val(lo_k, lo_v, descending=True)


def _gather16(x, idx):
    dnums = lax.GatherDimensionNumbers(
        offset_dims=(), collapsed_slice_dims=(0,), start_index_map=(0,))
    return lax.gather(x, idx[:, None], dnums, slice_sizes=(1,),
                      mode=lax.GatherScatterMode.PROMISE_IN_BOUNDS)


def _splat_last(x):
    return _gather16(x, jnp.full((LANES,), LANES - 1, jnp.int32))


def _lane_sum(m):
    lane_iota = lax.iota(jnp.int32, LANES)
    s = m
    for k in (8, 4, 2, 1):
        s = s + _gather16(s, lane_iota ^ k)
    return s


def _lane_any(m):
    lane_iota = lax.iota(jnp.int32, LANES)
    s = m.astype(jnp.int32)
    for k in (8, 4, 2, 1):
        s = s | _gather16(s, lane_iota ^ k)
    return s[0] > 0


def _tau_of(lk_ref):
    return _gather16(lk_ref[3], jnp.full((LANES,), 1, jnp.int32))


def _rebuild(lk_ref, lv_ref, pool_v, pool_i, cnt):
    lane_iota = lax.iota(jnp.int32, LANES)
    for j in range(POOL // LANES):
        off = j * LANES

        @pl.when(off < cnt)
        def _():
            pv = pool_v[pl.ds(off, LANES)]
            pi = pool_i[pl.ds(off, LANES)]
            valid = (lane_iota + off) < cnt
            pvm = jnp.where(valid, pv, NEG_SENT)
            sk, sv = plsc.sort_key_val(pvm, pi, descending=True)
            _merge_into_list(lk_ref, lv_ref, sk, sv)

    return _tau_of(lk_ref)


def _march_row(row, buf_ref, pool_v, pool_i, lk_ref, lv_ref,
               stag_v, stag_i, out_s_hbm, out_i_hbm):
    nvec = (jnp.maximum(row, 56) + 15) // 16
    ngroups = (nvec + 3) // 4
    lane_iota = lax.iota(jnp.int32, LANES)

    for j in range(4):
        lk_ref[j] = jnp.full((LANES,), NEG_SENT, jnp.float32)
        lv_ref[j] = jnp.zeros((LANES,), jnp.int32)

    for j in range(4):
        v = buf_ref[pl.ds(j * LANES, LANES)]
        sk, sv = plsc.sort_key_val(v, lane_iota + j * LANES, descending=True)
        _merge_into_list(lk_ref, lv_ref, sk, sv)
    tau0 = _tau_of(lk_ref)

    def group_body(g, carry):
        tau, cnt = carry
        base = g * 4 * LANES
        vs = [buf_ref[pl.ds(base + j * LANES, LANES)] for j in range(4)]
        ms = [v > tau for v in vs]
        anym = ms[0] | ms[1] | ms[2] | ms[3]

        def group_append():
            c = cnt
            for j in range(4):
                vj, mj = vs[j], ms[j]
                offj = base + j * LANES

                def append_j(c=c, vj=vj, mj=mj, offj=offj):
                    vm = jnp.where(mj, vj, NEG_SENT)
                    sk, sv = plsc.sort_key_val(vm, lane_iota + offj,
                                               descending=True)
                    slots = c + lane_iota
                    plsc.store_scatter(pool_v, [slots], sk)
                    plsc.store_scatter(pool_i, [slots], sv)
                    return c + _lane_sum(mj.astype(jnp.int32))[0]

                c = lax.cond(_lane_any(mj), append_j, lambda c=c: c)

            def rebuild_branch():
                return _rebuild(lk_ref, lv_ref, pool_v, pool_i, c), 0

            return lax.cond(c >= REBUILD_AT, rebuild_branch,
                            lambda: (tau, c))


        return lax.cond(_lane_any(anym), group_append, lambda: (tau, cnt))

    _, cnt = lax.fori_loop(1, ngroups, group_body, (tau0, jnp.int32(0)))
    _rebuild(lk_ref, lv_ref, pool_v, pool_i, cnt)

    for j in range(4):
        k = lk_ref[j]
        outv = jnp.where(k < -1.0e29, NEG_INF, k)
        stag_v[pl.ds(j * LANES, LANES)] = outv
        stag_i[pl.ds(j * LANES, LANES)] = lv_ref[j]
    pltpu.sync_copy(stag_v, out_s_hbm.at[row])
    pltpu.sync_copy(stag_i, out_i_hbm.at[row])


def _sc_topk_body(scores_hbm, out_s_hbm, out_i_hbm,
                  rowbuf_a, rowbuf_b, pool_v, pool_i, lk_ref, lv_ref,
                  stag_v, stag_i, sem):
    wid = lax.axis_index("s") * NC + lax.axis_index("c")
    rows_per_worker = N // NW

    pltpu.async_copy(scores_hbm.at[wid], rowbuf_a, sem)

    def pair_body(q, _):
        row_a = q * 2 * NW + wid
        row_b = row_a + NW
        pltpu.make_async_copy(scores_hbm.at[row_a], rowbuf_a, sem).wait()
        pltpu.async_copy(scores_hbm.at[row_b], rowbuf_b, sem)
        _march_row(row_a, rowbuf_a, pool_v, pool_i, lk_ref, lv_ref,
                   stag_v, stag_i, out_s_hbm, out_i_hbm)
        pltpu.make_async_copy(scores_hbm.at[row_b], rowbuf_b, sem).wait()

        @pl.when(q + 1 < rows_per_worker // 2)
        def _():
            pltpu.async_copy(scores_hbm.at[row_a + 2 * NW], rowbuf_a, sem)

        _march_row(row_b, rowbuf_b, pool_v, pool_i, lk_ref, lv_ref,
                   stag_v, stag_i, out_s_hbm, out_i_hbm)
        return 0

    lax.fori_loop(0, rows_per_worker // 2, pair_body, 0)


_sc_topk = functools.partial(
    pl.kernel,
    out_type=[
        jax.ShapeDtypeStruct((N, KPAD), jnp.float32),
        jax.ShapeDtypeStruct((N, KPAD), jnp.int32),
    ],
    mesh=plsc.VectorSubcoreMesh(core_axis_name="c", subcore_axis_name="s"),
    compiler_params=pltpu.CompilerParams(needs_layout_passes=False),
    scratch_types=[
        pltpu.VMEM((N,), jnp.float32),
        pltpu.VMEM((N,), jnp.float32),
        pltpu.VMEM((POOL,), jnp.float32),
        pltpu.VMEM((POOL,), jnp.int32),
        pltpu.VMEM((4, LANES), jnp.float32),
        pltpu.VMEM((4, LANES), jnp.int32),
        pltpu.VMEM((KPAD,), jnp.float32),
        pltpu.VMEM((KPAD,), jnp.int32),
        pltpu.SemaphoreType.DMA,
    ],
)(_sc_topk_body)


@jax.jit
def kernel(mentions, W, b, first):
    scores = _compute_scores(mentions, W, b)
    top_s, top_i = _sc_topk(scores)
    return top_s[:, :K], top_i[:, :K]


# speedup vs baseline: 7.3470x; 1.0267x over previous
import functools

import jax
import jax.numpy as jnp
from jax import lax
from jax.experimental import pallas as pl
from jax.experimental.pallas import tpu as pltpu
from jax.experimental.pallas import tpu_sc as plsc

N = 8192
D = 128
K = 50
KPAD = 64
BR = 256
BC = 256
MASK_BASE = -1.0e30
MASK_STEP = -1.0e26
NEG_SENT = -3.0e38
NEG_INF = float("-inf")

NC = 2
NS = 16
NW = NC * NS
LANES = 16
POOL = 128
REBUILD_AT = 64


def _scores_kernel(m_blk_ref, m_full_ref, w_ref, b_ref, out_ref):
    i = pl.program_id(0)

    w = lax.dot_general(
        m_blk_ref[...], w_ref[...],
        dimension_numbers=(((1,), (1,)), ((), ())),
        preferred_element_type=jnp.float32) + b_ref[...]

    row_ids = lax.broadcasted_iota(jnp.int32, (BR, BC), 0) + i * BR

    def col_block(c, _):
        m_c = m_full_ref[pl.ds(c * BC, BC), :]
        s = lax.dot_general(
            w, m_c,
            dimension_numbers=(((1,), (1,)), ((), ())),
            preferred_element_type=jnp.float32)
        col_ids = lax.broadcasted_iota(jnp.int32, (BR, BC), 1) + c * BC
        maskval = MASK_BASE + col_ids.astype(jnp.float32) * MASK_STEP
        out_ref[:, pl.ds(c * BC, BC)] = jnp.where(col_ids < row_ids, s,
                                                  maskval)
        return 0

    n_col_blocks = m_full_ref.shape[0] // BC
    lax.fori_loop(0, jnp.minimum(i + 2, n_col_blocks), col_block, 0)


def _compute_scores(mentions, W, b):
    n = mentions.shape[0]
    return pl.pallas_call(
        _scores_kernel,
        grid=(n // BR,),
        in_specs=[
            pl.BlockSpec((BR, D), lambda i: (i, 0)),
            pl.BlockSpec((n, D), lambda i: (0, 0)),
            pl.BlockSpec((D, D), lambda i: (0, 0)),
            pl.BlockSpec((1, D), lambda i: (0, 0)),
        ],
        out_specs=pl.BlockSpec((BR, n), lambda i: (i, 0)),
        out_shape=jax.ShapeDtypeStruct((n, n), jnp.float32),
        compiler_params=pltpu.CompilerParams(
            dimension_semantics=("arbitrary",)),
    )(mentions, mentions, W, b.reshape(1, D))


def _merge_into_list(lk_ref, lv_ref, a_k, a_v):
    for t in range(4):
        b_k = lk_ref[t]
        b_v = lv_ref[t]
        rb_k = lax.rev(b_k, (0,))
        rb_v = lax.rev(b_v, (0,))
        take_a = a_k >= rb_k
        hi_k = jnp.where(take_a, a_k, rb_k)
        hi_v = jnp.where(take_a, a_v, rb_v)
        lo_k = jnp.where(take_a, rb_k, a_k)
        lo_v = jnp.where(take_a, rb_v, a_v)
        nk, nv = plsc.sort_key_val(hi_k, hi_v, descending=True)
        lk_ref[t] = nk
        lv_ref[t] = nv
        a_k, a_v = plsc.sort_key_val(lo_k, lo_v, descending=True)


def _gather16(x, idx):
    dnums = lax.GatherDimensionNumbers(
        offset_dims=(), collapsed_slice_dims=(0,), start_index_map=(0,))
    return lax.gather(x, idx[:, None], dnums, slice_sizes=(1,),
                      mode=lax.GatherScatterMode.PROMISE_IN_BOUNDS)


def _splat_last(x):
    return _gather16(x, jnp.full((LANES,), LANES - 1, jnp.int32))


def _lane_sum(m):
    lane_iota = lax.iota(jnp.int32, LANES)
    s = m
    for k in (8, 4, 2, 1):
        s = s + _gather16(s, lane_iota ^ k)
    return s


def _lane_any(m):
    lane_iota = lax.iota(jnp.int32, LANES)
    s = m.astype(jnp.int32)
    for k in (8, 4, 2, 1):
        s = s | _gather16(s, lane_iota ^ k)
    return s[0] > 0


def _tau_of(lk_ref):
    return _gather16(lk_ref[3], jnp.full((LANES,), 1, jnp.int32))


def _rebuild(lk_ref, lv_ref, pool_v, pool_i, cnt):
    lane_iota = lax.iota(jnp.int32, LANES)
    for j in range(POOL // LANES):
        off = j * LANES

        @pl.when(off < cnt)
        def _():
            pv = pool_v[pl.ds(off, LANES)]
            pi = pool_i[pl.ds(off, LANES)]
            valid = (lane_iota + off) < cnt
            pvm = jnp.where(valid, pv, NEG_SENT)
            sk, sv = plsc.sort_key_val(pvm, pi, descending=True)
            _merge_into_list(lk_ref, lv_ref, sk, sv)

    return _tau_of(lk_ref)


def _march_row(row, buf_ref, pool_v, pool_i, lk_ref, lv_ref,
               stag_v, stag_i, out_s_hbm, out_i_hbm):
    nvec = (jnp.maximum(row, 56) + 15) // 16
    ngroups = (nvec + 3) // 4
    lane_iota = lax.iota(jnp.int32, LANES)

    for j in range(4):
        lk_ref[j] = jnp.full((LANES,), NEG_SENT, jnp.float32)
        lv_ref[j] = jnp.zeros((LANES,), jnp.int32)

    for j in range(4):
        v = buf_ref[pl.ds(j * LANES, LANES)]
        sk, sv = plsc.sort_key_val(v, lane_iota + j * LANES, descending=True)
        _merge_into_list(lk_ref, lv_ref, sk, sv)
    tau0 = _tau_of(lk_ref)

    def group_body(g, carry):
        tau, cnt = carry
        base = g * 4 * LANES
        vs = [buf_ref[pl.ds(base + j * LANES, LANES)] for j in range(4)]
        ms = [v > tau for v in vs]
        anym = ms[0] | ms[1] | ms[2] | ms[3]

        def group_append():
            c = cnt
            for j in range(4):
                vj, mj = vs[j], ms[j]
                offj = base + j * LANES

                def append_j(c=c, vj=vj, mj=mj, offj=offj):
                    vm = jnp.where(mj, vj, NEG_SENT)
                    sk, sv = plsc.sort_key_val(vm, lane_iota + offj,
                                               descending=True)
                    slots = c + lane_iota
                    plsc.store_scatter(pool_v, [slots], sk)
                    plsc.store_scatter(pool_i, [slots], sv)
                    return c + _lane_sum(mj.astype(jnp.int32))[0]

                c = lax.cond(_lane_any(mj), append_j, lambda c=c: c)

            def rebuild_branch():
                return _rebuild(lk_ref, lv_ref, pool_v, pool_i, c), 0

            return lax.cond(c >= REBUILD_AT, rebuild_branch,
                            lambda: (tau, c))


        return lax.cond(_lane_any(anym), group_append, lambda: (tau, cnt))

    _, cnt = lax.fori_loop(1, ngroups, group_body, (tau0, jnp.int32(0)))
    _rebuild(lk_ref, lv_ref, pool_v, pool_i, cnt)

    for j in range(4):
        k = lk_ref[j]
        outv = jnp.where(k < -1.0e29, NEG_INF, k)
        stag_v[pl.ds(j * LANES, LANES)] = outv
        stag_i[pl.ds(j * LANES, LANES)] = lv_ref[j]
    pltpu.sync_copy(stag_v, out_s_hbm.at[row])
    pltpu.sync_copy(stag_i, out_i_hbm.at[row])


def _sc_topk_body(scores_hbm, out_s_hbm, out_i_hbm,
                  rowbuf_a, rowbuf_b, pool_v, pool_i, lk_ref, lv_ref,
                  stag_v, stag_i, sem):
    wid = lax.axis_index("s") * NC + lax.axis_index("c")
    rows_per_worker = N // NW

    pltpu.async_copy(scores_hbm.at[wid], rowbuf_a, sem)

    def pair_body(q, _):
        row_a = q * 2 * NW + wid
        row_b = row_a + NW
        pltpu.make_async_copy(scores_hbm.at[row_a], rowbuf_a, sem).wait()
        pltpu.async_copy(scores_hbm.at[row_b], rowbuf_b, sem)
        _march_row(row_a, rowbuf_a, pool_v, pool_i, lk_ref, lv_ref,
                   stag_v, stag_i, out_s_hbm, out_i_hbm)
        pltpu.make_async_copy(scores_hbm.at[row_b], rowbuf_b, sem).wait()

        @pl.when(q + 1 < rows_per_worker // 2)
        def _():
            pltpu.async_copy(scores_hbm.at[row_a + 2 * NW], rowbuf_a, sem)

        _march_row(row_b, rowbuf_b, pool_v, pool_i, lk_ref, lv_ref,
                   stag_v, stag_i, out_s_hbm, out_i_hbm)
        return 0

    lax.fori_loop(0, rows_per_worker // 2, pair_body, 0)


_sc_topk = functools.partial(
    pl.kernel,
    out_type=[
        jax.ShapeDtypeStruct((N, KPAD), jnp.float32),
        jax.ShapeDtypeStruct((N, KPAD), jnp.int32),
    ],
    mesh=plsc.VectorSubcoreMesh(core_axis_name="c", subcore_axis_name="s"),
    compiler_params=pltpu.CompilerParams(needs_layout_passes=False),
    scratch_types=[
        pltpu.VMEM((N,), jnp.float32),
        pltpu.VMEM((N,), jnp.float32),
        pltpu.VMEM((POOL,), jnp.float32),
        pltpu.VMEM((POOL,), jnp.int32),
        pltpu.VMEM((4, LANES), jnp.float32),
        pltpu.VMEM((4, LANES), jnp.int32),
        pltpu.VMEM((KPAD,), jnp.float32),
        pltpu.VMEM((KPAD,), jnp.int32),
        pltpu.SemaphoreType.DMA,
    ],
)(_sc_topk_body)


@jax.jit
def kernel(mentions, W, b, first):
    scores = _compute_scores(mentions, W, b)
    top_s, top_i = _sc_topk(scores)
    return top_s[:, :K], top_i[:, :K]


# speedup vs baseline: 7.4791x; 1.0180x over previous
import functools

import jax
import jax.numpy as jnp
from jax import lax
from jax.experimental import pallas as pl
from jax.experimental.pallas import tpu as pltpu
from jax.experimental.pallas import tpu_sc as plsc

N = 8192
D = 128
K = 50
KPAD = 64
BR = 256
BC = 256
MASK_BASE = -1.0e30
MASK_STEP = -1.0e26
NEG_SENT = -3.0e38
NEG_INF = float("-inf")

NC = 2
NS = 16
NW = NC * NS
LANES = 16
POOL = 128
REBUILD_AT = 64


def _scores_kernel(m_blk_ref, m_full_ref, w_ref, b_ref, out_ref):
    i = pl.program_id(0)

    w = lax.dot_general(
        m_blk_ref[...], w_ref[...],
        dimension_numbers=(((1,), (1,)), ((), ())),
        preferred_element_type=jnp.float32) + b_ref[...]

    row_ids = lax.broadcasted_iota(jnp.int32, (BR, BC), 0) + i * BR

    def col_block(c, _):
        m_c = m_full_ref[pl.ds(c * BC, BC), :]
        s = lax.dot_general(
            w, m_c,
            dimension_numbers=(((1,), (1,)), ((), ())),
            preferred_element_type=jnp.float32)
        col_ids = lax.broadcasted_iota(jnp.int32, (BR, BC), 1) + c * BC
        maskval = MASK_BASE + col_ids.astype(jnp.float32) * MASK_STEP
        out_ref[:, pl.ds(c * BC, BC)] = jnp.where(col_ids < row_ids, s,
                                                  maskval)
        return 0

    n_col_blocks = m_full_ref.shape[0] // BC
    lax.fori_loop(0, jnp.minimum(i + 2, n_col_blocks), col_block, 0)


def _compute_scores(mentions, W, b):
    n = mentions.shape[0]
    return pl.pallas_call(
        _scores_kernel,
        grid=(n // BR,),
        in_specs=[
            pl.BlockSpec((BR, D), lambda i: (i, 0)),
            pl.BlockSpec((n, D), lambda i: (0, 0)),
            pl.BlockSpec((D, D), lambda i: (0, 0)),
            pl.BlockSpec((1, D), lambda i: (0, 0)),
        ],
        out_specs=pl.BlockSpec((BR, n), lambda i: (i, 0)),
        out_shape=jax.ShapeDtypeStruct((n, n), jnp.float32),
        compiler_params=pltpu.CompilerParams(
            dimension_semantics=("arbitrary",)),
    )(mentions, mentions, W, b.reshape(1, D))


def _merge_into_list(lk_ref, lv_ref, a_k, a_v):
    for t in range(4):
        b_k = lk_ref[t]
        b_v = lv_ref[t]
        rb_k = lax.rev(b_k, (0,))
        rb_v = lax.rev(b_v, (0,))
        take_a = (a_k > rb_k) | ((a_k == rb_k) & (a_v < rb_v))
        hi_k = jnp.where(take_a, a_k, rb_k)
        hi_v = jnp.where(take_a, a_v, rb_v)
        lo_k = jnp.where(take_a, rb_k, a_k)
        lo_v = jnp.where(take_a, rb_v, a_v)
        nk, nv = plsc.sort_key_val(hi_k, hi_v, descending=True)
        lk_ref[t] = nk
        lv_ref[t] = nv
        a_k, a_v = plsc.sort_key_val(lo_k, lo_v, descending=True)


def _gather16(x, idx):
    dnums = lax.GatherDimensionNumbers(
        offset_dims=(), collapsed_slice_dims=(0,), start_index_map=(0,))
    return lax.gather(x, idx[:, None], dnums, slice_sizes=(1,),
                      mode=lax.GatherScatterMode.PROMISE_IN_BOUNDS)


def _splat_last(x):
    return _gather16(x, jnp.full((LANES,), LANES - 1, jnp.int32))


def _lane_sum(m):
    lane_iota = lax.iota(jnp.int32, LANES)
    s = m
    for k in (8, 4, 2, 1):
        s = s + _gather16(s, lane_iota ^ k)
    return s


def _lane_any(m):
    lane_iota = lax.iota(jnp.int32, LANES)
    s = m.astype(jnp.int32)
    for k in (8, 4, 2, 1):
        s = s | _gather16(s, lane_iota ^ k)
    return s[0] > 0


def _tau_of(lk_ref):
    return _gather16(lk_ref[3], jnp.full((LANES,), 1, jnp.int32))


def _rebuild(lk_ref, lv_ref, pool_v, pool_i, cnt):
    lane_iota = lax.iota(jnp.int32, LANES)
    for j in range(POOL // LANES):
        off = j * LANES

        @pl.when(off < cnt)
        def _():
            pv = pool_v[pl.ds(off, LANES)]
            pi = pool_i[pl.ds(off, LANES)]
            valid = (lane_iota + off) < cnt
            pvm = jnp.where(valid, pv, NEG_SENT)
            sk, sv = plsc.sort_key_val(pvm, pi, descending=True)
            _merge_into_list(lk_ref, lv_ref, sk, sv)

    return _tau_of(lk_ref)


def _march_row(row, buf_ref, pool_v, pool_i, lk_ref, lv_ref,
               stag_v, stag_i, out_s_hbm, out_i_hbm):
    nvec = (jnp.maximum(row, 56) + 15) // 16
    ngroups = (nvec + 3) // 4
    lane_iota = lax.iota(jnp.int32, LANES)

    for j in range(4):
        lk_ref[j] = jnp.full((LANES,), NEG_SENT, jnp.float32)
        lv_ref[j] = jnp.zeros((LANES,), jnp.int32)

    for j in range(4):
        v = buf_ref[pl.ds(j * LANES, LANES)]
        sk, sv = plsc.sort_key_val(v, lane_iota + j * LANES, descending=True)
        _merge_into_list(lk_ref, lv_ref, sk, sv)
    tau0 = _tau_of(lk_ref)

    def group_body(g, carry):
        tau, cnt = carry
        base = g * 4 * LANES
        vs = [buf_ref[pl.ds(base + j * LANES, LANES)] for j in range(4)]
        ms = [v > tau for v in vs]
        anym = ms[0] | ms[1] | ms[2] | ms[3]

        def group_append():
            c = cnt
            for j in range(4):
                vj, mj = vs[j], ms[j]
                offj = base + j * LANES

                def append_j(c=c, vj=vj, mj=mj, offj=offj):
                    vm = jnp.where(mj, vj, NEG_SENT)
                    sk, sv = plsc.sort_key_val(vm, lane_iota + offj,
                                               descending=True)
                    slots = c + lane_iota
                    plsc.store_scatter(pool_v, [slots], sk)
                    plsc.store_scatter(pool_i, [slots], sv)
                    return c + _lane_sum(mj.astype(jnp.int32))[0]

                c = lax.cond(_lane_any(mj), append_j, lambda c=c: c)

            def rebuild_branch():
                return _rebuild(lk_ref, lv_ref, pool_v, pool_i, c), 0

            return lax.cond(c >= REBUILD_AT, rebuild_branch,
                            lambda: (tau, c))


        return lax.cond(_lane_any(anym), group_append, lambda: (tau, cnt))

    _, cnt = lax.fori_loop(1, ngroups, group_body, (tau0, jnp.int32(0)))
    _rebuild(lk_ref, lv_ref, pool_v, pool_i, cnt)

    for j in range(4):
        k = lk_ref[j]
        outv = jnp.where(k < -1.0e29, NEG_INF, k)
        stag_v[pl.ds(j * LANES, LANES)] = outv
        stag_i[pl.ds(j * LANES, LANES)] = lv_ref[j]
    pltpu.sync_copy(stag_v, out_s_hbm.at[row])
    pltpu.sync_copy(stag_i, out_i_hbm.at[row])


def _sc_topk_body(scores_hbm, out_s_hbm, out_i_hbm,
                  rowbuf_a, rowbuf_b, pool_v, pool_i, lk_ref, lv_ref,
                  stag_v, stag_i, sem):
    wid = lax.axis_index("s") * NC + lax.axis_index("c")
    rows_per_worker = N // NW

    pltpu.async_copy(scores_hbm.at[wid], rowbuf_a, sem)

    def pair_body(q, _):
        row_a = q * 2 * NW + wid
        row_b = row_a + NW
        pltpu.make_async_copy(scores_hbm.at[row_a], rowbuf_a, sem).wait()
        pltpu.async_copy(scores_hbm.at[row_b], rowbuf_b, sem)
        _march_row(row_a, rowbuf_a, pool_v, pool_i, lk_ref, lv_ref,
                   stag_v, stag_i, out_s_hbm, out_i_hbm)
        pltpu.make_async_copy(scores_hbm.at[row_b], rowbuf_b, sem).wait()

        @pl.when(q + 1 < rows_per_worker // 2)
        def _():
            pltpu.async_copy(scores_hbm.at[row_a + 2 * NW], rowbuf_a, sem)

        _march_row(row_b, rowbuf_b, pool_v, pool_i, lk_ref, lv_ref,
                   stag_v, stag_i, out_s_hbm, out_i_hbm)
        return 0

    lax.fori_loop(0, rows_per_worker // 2, pair_body, 0)


_sc_topk = functools.partial(
    pl.kernel,
    out_type=[
        jax.ShapeDtypeStruct((N, KPAD), jnp.float32),
        jax.ShapeDtypeStruct((N, KPAD), jnp.int32),
    ],
    mesh=plsc.VectorSubcoreMesh(core_axis_name="c", subcore_axis_name="s"),
    compiler_params=pltpu.CompilerParams(needs_layout_passes=False),
    scratch_types=[
        pltpu.VMEM((N,), jnp.float32),
        pltpu.VMEM((N,), jnp.float32),
        pltpu.VMEM((POOL,), jnp.float32),
        pltpu.VMEM((POOL,), jnp.int32),
        pltpu.VMEM((4, LANES), jnp.float32),
        pltpu.VMEM((4, LANES), jnp.int32),
        pltpu.VMEM((KPAD,), jnp.float32),
        pltpu.VMEM((KPAD,), jnp.int32),
        pltpu.SemaphoreType.DMA,
    ],
)(_sc_topk_body)


@jax.jit
def kernel(mentions, W, b, first):
    scores = _compute_scores(mentions, W, b)
    top_s, top_i = _sc_topk(scores)
    return top_s[:, :K], top_i[:, :K]
